# baseline (speedup 1.0000x reference)
import functools
import jax
import jax.numpy as jnp
from jax import lax
from jax.experimental import pallas as pl
from jax.experimental.pallas import tpu as pltpu
from jax.experimental.pallas import tpu_sc as plsc

B, N, F = 16, 4096, 512
E = 64
HI = 2048
TE = E // 2
LANES = 16
CPR = F // LANES

_mesh = plsc.VectorSubcoreMesh(core_axis_name="c", subcore_axis_name="s")


@functools.partial(
    pl.kernel,
    mesh=_mesh,
    out_type=jax.ShapeDtypeStruct((B, E, F), jnp.float32),
    scratch_types=[
        pltpu.VMEM((TE, F), jnp.float32),
        pltpu.VMEM((TE, F), jnp.float32),
        pltpu.SemaphoreType.DMA,
    ],
)
def _sc_midpoints(x_hbm, nv_hbm, lo_v, hi_v, sem):
    cid = lax.axis_index("c")
    sid = lax.axis_index("s")
    b = sid
    t0 = cid * TE

    ld_lo = pltpu.make_async_copy(x_hbm.at[b, pl.ds(t0, TE), :], lo_v, sem)
    ld_hi = pltpu.make_async_copy(x_hbm.at[b, pl.ds(HI + t0, TE), :], hi_v, sem)
    ld_lo.start()
    ld_hi.start()
    ld_lo.wait()
    ld_hi.wait()

    def _row(r, carry):
        for ci in range(CPR):
            cc = ci * LANES
            lo_v[r, pl.ds(cc, LANES)] = 0.5 * (
                lo_v[r, pl.ds(cc, LANES)] + hi_v[r, pl.ds(cc, LANES)]
            )
        return carry

    lax.fori_loop(0, TE, _row, 0)

    st = pltpu.make_async_copy(lo_v, nv_hbm.at[b, pl.ds(t0, TE), :], sem)
    st.start()
    st.wait()


RB = 2048


def _copy_body(x_ref, out_ref):
    out_ref[...] = x_ref[...]


def _merge_body(nv_ref, out1_ref, out_ref):
    out_ref[...] = nv_ref[...]


def kernel(inputs):
    nv = _sc_midpoints(inputs)
    out1 = pl.pallas_call(
        _copy_body,
        grid=(B, N // RB),
        in_specs=[pl.BlockSpec((1, RB, F), lambda b, j: (b, j, 0))],
        out_specs=pl.BlockSpec((1, RB, F), lambda b, j: (b, j, 0)),
        out_shape=jax.ShapeDtypeStruct((B, N + E, F), inputs.dtype),
    )(inputs)
    return pl.pallas_call(
        _merge_body,
        grid=(B,),
        in_specs=[
            pl.BlockSpec((1, E, F), lambda b: (b, 0, 0)),
            pl.BlockSpec((1, E, F), lambda b: (b, 0, 0)),
        ],
        out_specs=pl.BlockSpec((1, E, F), lambda b: (b, N // E, 0)),
        out_shape=jax.ShapeDtypeStruct((B, N + E, F), inputs.dtype),
        input_output_aliases={1: 0},
    )(nv, out1)


# speedup vs baseline: 1.0330x; 1.0330x over previous
import functools
import jax
import jax.numpy as jnp
from jax import lax
from jax.experimental import pallas as pl
from jax.experimental.pallas import tpu as pltpu
from jax.experimental.pallas import tpu_sc as plsc

B, N, F = 16, 4096, 512
E = 64
HI = 2048
TE = E // 2
LANES = 16
CPR = F // LANES

_mesh = plsc.VectorSubcoreMesh(core_axis_name="c", subcore_axis_name="s")


@functools.partial(
    pl.kernel,
    mesh=_mesh,
    out_type=jax.ShapeDtypeStruct((B, E, F), jnp.float32),
    scratch_types=[
        pltpu.VMEM((TE, F), jnp.float32),
        pltpu.VMEM((TE, F), jnp.float32),
        pltpu.SemaphoreType.DMA,
    ],
)
def _sc_midpoints(x_hbm, nv_hbm, lo_v, hi_v, sem):
    cid = lax.axis_index("c")
    sid = lax.axis_index("s")
    b = sid
    t0 = cid * TE

    ld_lo = pltpu.make_async_copy(x_hbm.at[b, pl.ds(t0, TE), :], lo_v, sem)
    ld_hi = pltpu.make_async_copy(x_hbm.at[b, pl.ds(HI + t0, TE), :], hi_v, sem)
    ld_lo.start()
    ld_hi.start()
    ld_lo.wait()
    ld_hi.wait()

    def _row(r, carry):
        for ci in range(CPR):
            cc = ci * LANES
            lo_v[r, pl.ds(cc, LANES)] = 0.5 * (
                lo_v[r, pl.ds(cc, LANES)] + hi_v[r, pl.ds(cc, LANES)]
            )
        return carry

    lax.fori_loop(0, TE, _row, 0)

    st = pltpu.make_async_copy(lo_v, nv_hbm.at[b, pl.ds(t0, TE), :], sem)
    st.start()
    st.wait()


RB = 2080
NBLK = (N + E) // RB
TAIL_COPY = N - (NBLK - 1) * RB


def _tc_body(x_ref, nv_ref, out_ref):
    j = pl.program_id(1)

    @pl.when(j < NBLK - 1)
    def _copy():
        out_ref[...] = x_ref[...]

    @pl.when(j == NBLK - 1)
    def _tail():
        out_ref[0, :TAIL_COPY, :] = x_ref[0, :TAIL_COPY, :]
        out_ref[0, TAIL_COPY:, :] = nv_ref[0]


def kernel(inputs):
    nv = _sc_midpoints(inputs)
    return pl.pallas_call(
        _tc_body,
        grid=(B, NBLK),
        in_specs=[
            pl.BlockSpec((1, RB, F), lambda b, j: (b, j, 0)),
            pl.BlockSpec((1, E, F), lambda b, j: (b, 0, 0)),
        ],
        out_specs=pl.BlockSpec((1, RB, F), lambda b, j: (b, j, 0)),
        out_shape=jax.ShapeDtypeStruct((B, N + E, F), inputs.dtype),
    )(inputs, nv)
